# baseline (speedup 1.0000x reference)
import numpy as np
import jax
import jax.numpy as jnp
from jax.experimental import pallas as pl
from jax.experimental.pallas import tpu as pltpu

B = 16
N = 4096
NDIR = 8
SL = 3
NM = 12
DIM = 128
DIMS = [8, 16, 16, 32, 32, 64, 64, 128, 128, 128, 128, 128, 128]
OFFS = [0]
for _j in range(1, NM + 1):
    OFFS.append(OFFS[-1] + (N >> _j))
DN0 = (((0,), (0,)), ((), ()))
DN1 = (((1,), (1,)), ((), ()))


def _prelu(h, a):
    return jnp.where(h >= 0, h, a * h)


def _lvl(A, vrow, wf, bb, dmap_s, drev_s, alpha, idim, odim, sp, W):
    Ash = jnp.concatenate([A[:, sp:], A[:, :sp]], axis=1)
    cat = jnp.concatenate([A, Ash, jnp.ones((1, W), jnp.float32)], axis=0)
    h = jnp.zeros((odim, W), jnp.float32)
    for d in range(NDIR):
        e = dmap_s[d]
        Wd = wf[pl.ds(e * (2 * idim), 2 * idim), :]
        Wsw = jnp.concatenate([Wd[idim:], Wd[:idim]], axis=0)
        Wsel = jnp.where(drev_s[d] == 1, Wsw, Wd)
        brow = bb[pl.ds(e, 1), :]
        Waug = jnp.concatenate([Wsel, brow], axis=0)
        Hd = jax.lax.dot_general(Waug, cat, DN0,
                                 preferred_element_type=jnp.float32)
        h = h + (vrow == d).astype(jnp.float32) * Hd
    return _prelu(h, alpha)


def _samp(h, smp, wup, bup, wmg, bmg, a_up, a_mg, W):
    ones = jnp.ones((1, W), jnp.float32)
    wupa = jnp.concatenate([wup[...], bup[...]], axis=0)
    smp2 = _prelu(jax.lax.dot_general(
        wupa, jnp.concatenate([smp, ones], axis=0), DN0,
        preferred_element_type=jnp.float32), a_up)
    wmga = jnp.concatenate([wmg[...], bmg[...]], axis=0)
    hcat = jnp.concatenate([h, smp2, ones], axis=0)
    return _prelu(jax.lax.dot_general(
        wmga, hcat, DN0, preferred_element_type=jnp.float32), a_mg)


def _body_a(x_ref, vx_ref, wl_ref, w1, b1, w2, b2, dmap_s, drev_s, al,
            out_ref):
    W = 65536
    cat0 = jnp.concatenate([x_ref[...], jnp.ones((1, W), jnp.float32)],
                           axis=0)
    a0 = _prelu(jax.lax.dot_general(wl_ref[...], cat0, DN0,
                                    preferred_element_type=jnp.float32),
                al[0])
    out_ref[0:8, :] = a0
    a1 = _lvl(a0, vx_ref[0:1, :], w1, b1, dmap_s, drev_s, al[1], 8, 16, 1, W)
    out_ref[8:24, :] = a1
    a2 = _lvl(a1, vx_ref[1:2, :], w2, b2, dmap_s, drev_s, al[2], 16, 16, 2, W)
    out_ref[24:40, :] = a2


def _body_b(ca4_ref, vx_ref, w3, b3, up3w, up3b, mg3w, mg3b,
            w4, b4, up4w, up4b, mg4w, mg4b, dmap_s, drev_s, al, out_ref):
    W = 16384
    a2 = ca4_ref[24:40, :]
    a3 = _lvl(a2, vx_ref[0:1, :], w3, b3, dmap_s, drev_s, al[0], 16, 32, 1, W)
    a3 = _samp(a3, ca4_ref[0:8, :], up3w, up3b, mg3w, mg3b, al[1], al[2], W)
    out_ref[0:32, :] = a3
    a4 = _lvl(a3, vx_ref[1:2, :], w4, b4, dmap_s, drev_s, al[3], 32, 32, 2, W)
    a4 = _samp(a4, ca4_ref[8:24, :], up4w, up4b, mg4w, mg4b, al[4], al[5], W)
    out_ref[32:64, :] = a4


def _body_c(cb4_ref, ca16_ref, vx_ref,
            w5, b5, up5w, up5b, mg5w, mg5b,
            w6, b6, up6w, up6b, mg6w, mg6b,
            w7, b7, up7w, up7b, mg7w, mg7b, dmap_s, drev_s, al, out_ref):
    W = 4096
    a4 = cb4_ref[32:64, :]
    a5 = _lvl(a4, vx_ref[0:1, :], w5, b5, dmap_s, drev_s, al[0], 32, 64, 1, W)
    a5 = _samp(a5, ca16_ref[24:40, :], up5w, up5b, mg5w, mg5b,
               al[1], al[2], W)
    out_ref[0:64, :] = a5
    a6 = _lvl(a5, vx_ref[1:2, :], w6, b6, dmap_s, drev_s, al[3], 64, 64, 2, W)
    a6 = _samp(a6, cb4_ref[0:32, :], up6w, up6b, mg6w, mg6b, al[4], al[5], W)
    out_ref[64:128, :] = a6
    a7 = _lvl(a6, vx_ref[2:3, :], w7, b7, dmap_s, drev_s, al[6], 64, 128, 4,
              W)
    a7 = _samp(a7, cb4_ref[32:64, :], up7w, up7b, mg7w, mg7b,
               al[7], al[8], W)
    out_ref[128:256, :] = a7


def _body_d(*refs):
    W = 512
    out_ref = refs[-1]
    al = refs[-2]
    drev_s = refs[-3]
    dmap_s = refs[-4]
    cc8_ref, vx_ref, e_ref = refs[0:3]
    wrefs = refs[3:-4]
    A = cc8_ref[128:256, :]
    saved = {}
    for t in range(5):
        j = 8 + t
        w, bb, upw, upb, mgw, mgb = wrefs[6 * t:6 * t + 6]
        A = _lvl(A, vx_ref[t:t + 1, :], w, bb, dmap_s, drev_s, al[3 * t],
                 128, 128, 2 ** t, W)
        if j == 8:
            smp = cc8_ref[0:64, :]
        elif j == 9:
            smp = cc8_ref[64:128, :]
        elif j == 10:
            smp = cc8_ref[128:256, :]
        else:
            smp = saved[j - SL]
        A = _samp(A, smp, upw, upb, mgw, mgb, al[3 * t + 1], al[3 * t + 2],
                  W)
        saved[j] = A
    out_ref[...] = jax.lax.dot_general(e_ref[...], A, DN1,
                                       preferred_element_type=jnp.float32)


def _vexp_idx(levels, width):
    wb = width // B
    c = np.arange(width)
    b = c // wb
    m = c % wb
    rows = []
    for j in levels:
        s = wb // (N >> j)
        rows.append(b * (N - 1) + OFFS[j - 1] + m // s)
    return jnp.asarray(np.stack(rows).astype(np.int32))


def kernel(points, vec_flat, dmap, drev, axisperm, axissgn, params):
    f32 = jnp.float32
    smem = pl.BlockSpec(memory_space=pltpu.SMEM)
    vmem = pl.BlockSpec(memory_space=pltpu.VMEM)

    def vspecs(k):
        return [vmem for _ in range(k)]

    Wl, bl, al_leaf = params["leaf"]
    P = (axisperm[None, :] == jnp.arange(3)[:, None]).astype(f32)
    wl_aug = jnp.concatenate([(P * axissgn[None, :]) @ Wl, bl[None, :]],
                             axis=0)
    x_t = points.reshape(B * N, 3).T

    vflat = vec_flat.reshape(B * (N - 1))
    dmap_i = dmap.astype(jnp.int32)
    drev_i = drev.astype(jnp.int32)

    def wargs(j):
        Wm, bbm, _ = params["merge"][j - 1]
        return [Wm.reshape(NDIR * 2 * DIMS[j - 1], DIMS[j]), bbm]

    def sargs(j):
        wup, bup, _ = params["samp_up"][str(j)]
        wmg, bmg, _ = params["samp_mg"][str(j)]
        return [wup, bup[None, :], wmg, bmg[None, :]]

    def alphas(js, lead=None):
        out = [] if lead is None else list(lead)
        for j in js:
            out.append(params["merge"][j - 1][2])
            if j >= SL:
                out.append(params["samp_up"][str(j)][2])
                out.append(params["samp_mg"][str(j)][2])
        return jnp.stack(out).astype(f32)

    vx_a = jnp.take(vflat, _vexp_idx([1, 2], 65536))
    comb_a = pl.pallas_call(
        _body_a,
        out_shape=jax.ShapeDtypeStruct((40, 65536), f32),
        in_specs=vspecs(7) + [smem, smem, smem],
        out_specs=vmem,
    )(x_t, vx_a, wl_aug, *wargs(1), *wargs(2), dmap_i, drev_i,
      alphas([1, 2], lead=[al_leaf]))

    ca4 = comb_a[:, ::4]
    ca16 = comb_a[:, ::16]

    vx_b = jnp.take(vflat, _vexp_idx([3, 4], 16384))
    comb_b = pl.pallas_call(
        _body_b,
        out_shape=jax.ShapeDtypeStruct((64, 16384), f32),
        in_specs=vspecs(14) + [smem, smem, smem],
        out_specs=vmem,
    )(ca4, vx_b, *wargs(3), *sargs(3), *wargs(4), *sargs(4),
      dmap_i, drev_i, alphas([3, 4]))

    cb4 = comb_b[:, ::4]

    vx_c = jnp.take(vflat, _vexp_idx([5, 6, 7], 4096))
    comb_c = pl.pallas_call(
        _body_c,
        out_shape=jax.ShapeDtypeStruct((256, 4096), f32),
        in_specs=vspecs(21) + [smem, smem, smem],
        out_specs=vmem,
    )(cb4, ca16, vx_c, *wargs(5), *sargs(5), *wargs(6), *sargs(6),
      *wargs(7), *sargs(7), dmap_i, drev_i, alphas([5, 6, 7]))

    cc8 = comb_c[:, ::8]

    vx_d = jnp.take(vflat, _vexp_idx([8, 9, 10, 11, 12], 512))
    e_sel = jnp.asarray(
        (np.arange(512)[None, :] == (np.arange(B) * 32)[:, None])
        .astype(np.float32))
    wd = []
    for j in range(8, NM + 1):
        wd += wargs(j) + sargs(j)
    out = pl.pallas_call(
        _body_d,
        out_shape=jax.ShapeDtypeStruct((B, DIM), f32),
        in_specs=vspecs(3 + len(wd)) + [smem, smem, smem],
        out_specs=vmem,
    )(cc8, vx_d, e_sel, *wd, dmap_i, drev_i,
      alphas([8, 9, 10, 11, 12]))
    return out


# speedup vs baseline: 3.9249x; 3.9249x over previous
import numpy as np
import jax
import jax.numpy as jnp
from jax.experimental import pallas as pl
from jax.experimental.pallas import tpu as pltpu

B = 16
N = 4096
NDIR = 8
SL = 3
NM = 12
DIM = 128
DIMS = [8, 16, 16, 32, 32, 64, 64, 128, 128, 128, 128, 128, 128]
OFFS = [0]
for _j in range(1, NM + 1):
    OFFS.append(OFFS[-1] + (N >> _j))
DN0 = (((0,), (0,)), ((), ()))
DN1 = (((1,), (1,)), ((), ()))


def _prelu(h, a):
    return jnp.where(h >= 0, h, a * h)


def _lvl(A, vrow, wf, bb, dmap_s, drev_s, alpha, idim, odim, sp, W):
    Ash = jnp.concatenate([A[:, sp:], A[:, :sp]], axis=1)
    cat = jnp.concatenate([A, Ash, jnp.ones((1, W), jnp.float32)], axis=0)
    h = jnp.zeros((odim, W), jnp.float32)
    for d in range(NDIR):
        e = dmap_s[d]
        Wd = wf[pl.ds(e * (2 * idim), 2 * idim), :]
        Wsw = jnp.concatenate([Wd[idim:], Wd[:idim]], axis=0)
        Wsel = jnp.where(drev_s[d] == 1, Wsw, Wd)
        brow = bb[pl.ds(e, 1), :]
        Waug = jnp.concatenate([Wsel, brow], axis=0)
        Hd = jax.lax.dot_general(Waug, cat, DN0,
                                 preferred_element_type=jnp.float32)
        h = h + (vrow == d).astype(jnp.float32) * Hd
    return _prelu(h, alpha)


def _samp(h, smp, wup, bup, wmg, bmg, a_up, a_mg, W):
    ones = jnp.ones((1, W), jnp.float32)
    wupa = jnp.concatenate([wup[...], bup[...]], axis=0)
    smp2 = _prelu(jax.lax.dot_general(
        wupa, jnp.concatenate([smp, ones], axis=0), DN0,
        preferred_element_type=jnp.float32), a_up)
    wmga = jnp.concatenate([wmg[...], bmg[...]], axis=0)
    hcat = jnp.concatenate([h, smp2, ones], axis=0)
    return _prelu(jax.lax.dot_general(
        wmga, hcat, DN0, preferred_element_type=jnp.float32), a_mg)


def _body_a(x_ref, vx_ref, wl_ref, w1, b1, w2, b2, dmap_s, drev_s, al,
            out_ref):
    W = 65536
    cat0 = jnp.concatenate([x_ref[...], jnp.ones((1, W), jnp.float32)],
                           axis=0)
    a0 = _prelu(jax.lax.dot_general(wl_ref[...], cat0, DN0,
                                    preferred_element_type=jnp.float32),
                al[0])
    out_ref[0:8, :] = a0
    a1 = _lvl(a0, vx_ref[0:1, :], w1, b1, dmap_s, drev_s, al[1], 8, 16, 1, W)
    out_ref[8:24, :] = a1
    a2 = _lvl(a1, vx_ref[1:2, :], w2, b2, dmap_s, drev_s, al[2], 16, 16, 2, W)
    out_ref[24:40, :] = a2


def _body_b(ca4_ref, vx_ref, w3, b3, up3w, up3b, mg3w, mg3b,
            w4, b4, up4w, up4b, mg4w, mg4b, dmap_s, drev_s, al, out_ref):
    W = 16384
    a2 = ca4_ref[24:40, :]
    a3 = _lvl(a2, vx_ref[0:1, :], w3, b3, dmap_s, drev_s, al[0], 16, 32, 1, W)
    a3 = _samp(a3, ca4_ref[0:8, :], up3w, up3b, mg3w, mg3b, al[1], al[2], W)
    out_ref[0:32, :] = a3
    a4 = _lvl(a3, vx_ref[1:2, :], w4, b4, dmap_s, drev_s, al[3], 32, 32, 2, W)
    a4 = _samp(a4, ca4_ref[8:24, :], up4w, up4b, mg4w, mg4b, al[4], al[5], W)
    out_ref[32:64, :] = a4


def _body_c(cb4_ref, ca16_ref, vx_ref,
            w5, b5, up5w, up5b, mg5w, mg5b,
            w6, b6, up6w, up6b, mg6w, mg6b,
            w7, b7, up7w, up7b, mg7w, mg7b, dmap_s, drev_s, al, out_ref):
    W = 4096
    a4 = cb4_ref[32:64, :]
    a5 = _lvl(a4, vx_ref[0:1, :], w5, b5, dmap_s, drev_s, al[0], 32, 64, 1, W)
    a5 = _samp(a5, ca16_ref[24:40, :], up5w, up5b, mg5w, mg5b,
               al[1], al[2], W)
    out_ref[0:64, :] = a5
    a6 = _lvl(a5, vx_ref[1:2, :], w6, b6, dmap_s, drev_s, al[3], 64, 64, 2, W)
    a6 = _samp(a6, cb4_ref[0:32, :], up6w, up6b, mg6w, mg6b, al[4], al[5], W)
    out_ref[64:128, :] = a6
    a7 = _lvl(a6, vx_ref[2:3, :], w7, b7, dmap_s, drev_s, al[6], 64, 128, 4,
              W)
    a7 = _samp(a7, cb4_ref[32:64, :], up7w, up7b, mg7w, mg7b,
               al[7], al[8], W)
    out_ref[128:256, :] = a7


def _body_d(*refs):
    W = 512
    out_ref = refs[-1]
    al = refs[-2]
    drev_s = refs[-3]
    dmap_s = refs[-4]
    cc8_ref, vx_ref, e_ref = refs[0:3]
    wrefs = refs[3:-4]
    A = cc8_ref[128:256, :]
    saved = {}
    for t in range(5):
        j = 8 + t
        w, bb, upw, upb, mgw, mgb = wrefs[6 * t:6 * t + 6]
        A = _lvl(A, vx_ref[t:t + 1, :], w, bb, dmap_s, drev_s, al[3 * t],
                 128, 128, 2 ** t, W)
        if j == 8:
            smp = cc8_ref[0:64, :]
        elif j == 9:
            smp = cc8_ref[64:128, :]
        elif j == 10:
            smp = cc8_ref[128:256, :]
        else:
            smp = saved[j - SL]
        A = _samp(A, smp, upw, upb, mgw, mgb, al[3 * t + 1], al[3 * t + 2],
                  W)
        saved[j] = A
    out_ref[...] = jax.lax.dot_general(e_ref[...], A, DN1,
                                       preferred_element_type=jnp.float32)


def _vexp_dense(vec_flat, levels, width):
    wb = width // B
    rows = []
    for j in levels:
        n = N >> j
        v = vec_flat[:, OFFS[j - 1]:OFFS[j - 1] + n]
        rows.append(jnp.broadcast_to(v[:, :, None],
                                     (B, n, wb // n)).reshape(1, width))
    return jnp.concatenate(rows, axis=0)


def kernel(points, vec_flat, dmap, drev, axisperm, axissgn, params):
    f32 = jnp.float32
    smem = pl.BlockSpec(memory_space=pltpu.SMEM)
    vmem = pl.BlockSpec(memory_space=pltpu.VMEM)

    def vspecs(k):
        return [vmem for _ in range(k)]

    Wl, bl, al_leaf = params["leaf"]
    P = (axisperm[None, :] == jnp.arange(3)[:, None]).astype(f32)
    wl_aug = jnp.concatenate([(P * axissgn[None, :]) @ Wl, bl[None, :]],
                             axis=0)
    x_t = points.reshape(B * N, 3).T

    dmap_i = dmap.astype(jnp.int32)
    drev_i = drev.astype(jnp.int32)

    def wargs(j):
        Wm, bbm, _ = params["merge"][j - 1]
        return [Wm.reshape(NDIR * 2 * DIMS[j - 1], DIMS[j]), bbm]

    def sargs(j):
        wup, bup, _ = params["samp_up"][str(j)]
        wmg, bmg, _ = params["samp_mg"][str(j)]
        return [wup, bup[None, :], wmg, bmg[None, :]]

    def alphas(js, lead=None):
        out = [] if lead is None else list(lead)
        for j in js:
            out.append(params["merge"][j - 1][2])
            if j >= SL:
                out.append(params["samp_up"][str(j)][2])
                out.append(params["samp_mg"][str(j)][2])
        return jnp.stack(out).astype(f32)

    vx_a = _vexp_dense(vec_flat, [1, 2], 65536)
    comb_a = pl.pallas_call(
        _body_a,
        out_shape=jax.ShapeDtypeStruct((40, 65536), f32),
        in_specs=vspecs(7) + [smem, smem, smem],
        out_specs=vmem,
    )(x_t, vx_a, wl_aug, *wargs(1), *wargs(2), dmap_i, drev_i,
      alphas([1, 2], lead=[al_leaf]))

    ca4 = comb_a[:, ::4]
    ca16 = comb_a[:, ::16]

    vx_b = _vexp_dense(vec_flat, [3, 4], 16384)
    comb_b = pl.pallas_call(
        _body_b,
        out_shape=jax.ShapeDtypeStruct((64, 16384), f32),
        in_specs=vspecs(14) + [smem, smem, smem],
        out_specs=vmem,
    )(ca4, vx_b, *wargs(3), *sargs(3), *wargs(4), *sargs(4),
      dmap_i, drev_i, alphas([3, 4]))

    cb4 = comb_b[:, ::4]

    vx_c = _vexp_dense(vec_flat, [5, 6, 7], 4096)
    comb_c = pl.pallas_call(
        _body_c,
        out_shape=jax.ShapeDtypeStruct((256, 4096), f32),
        in_specs=vspecs(21) + [smem, smem, smem],
        out_specs=vmem,
    )(cb4, ca16, vx_c, *wargs(5), *sargs(5), *wargs(6), *sargs(6),
      *wargs(7), *sargs(7), dmap_i, drev_i, alphas([5, 6, 7]))

    cc8 = comb_c[:, ::8]

    vx_d = _vexp_dense(vec_flat, [8, 9, 10, 11, 12], 512)
    e_sel = jnp.asarray(
        (np.arange(512)[None, :] == (np.arange(B) * 32)[:, None])
        .astype(np.float32))
    wd = []
    for j in range(8, NM + 1):
        wd += wargs(j) + sargs(j)
    out = pl.pallas_call(
        _body_d,
        out_shape=jax.ShapeDtypeStruct((B, DIM), f32),
        in_specs=vspecs(3 + len(wd)) + [smem, smem, smem],
        out_specs=vmem,
    )(cc8, vx_d, e_sel, *wd, dmap_i, drev_i,
      alphas([8, 9, 10, 11, 12]))
    return out
